# baseline (speedup 1.0000x reference)
import functools

import jax
import jax.numpy as jnp
from jax import lax
from jax.experimental import pallas as pl
from jax.experimental.pallas import tpu as pltpu
from jax.experimental.pallas import tpu_sc as plsc

B, S, D, E, H = 1, 2048, 768, 8, 1024
T = B * S
TILE = 128
MAX_TILES = 24
P = MAX_TILES * TILE
KH = 1
HC = H // KH
CHUNK = 128
NCH = T // CHUNK

_INV_SQRT2 = 0.7071067811865476


def _gate_body(x_ref, wg_ref, bg_ref, pos_ref, meta_ref):
    xf = x_ref[0]
    scores = jnp.dot(xf, wg_ref[...],
                     preferred_element_type=jnp.float32) + bg_ref[...]
    m = jnp.max(scores, axis=1, keepdims=True)
    eidx = lax.broadcasted_iota(jnp.int32, (T, E), 1)
    sel = jnp.min(jnp.where(scores >= m, eidx, E), axis=1,
                  keepdims=True)
    onehot = (eidx == sel).astype(jnp.float32)
    counts = jnp.sum(onehot, axis=0, keepdims=True)
    ntiles = jnp.floor((counts + (TILE - 1)) * (1.0 / TILE))
    lt = (eidx < sel).astype(jnp.float32)
    rowoff = TILE * jnp.sum(ntiles * lt, axis=1, keepdims=True)
    tri = (lax.broadcasted_iota(jnp.int32, (CHUNK, CHUNK), 1)
           <= lax.broadcasted_iota(jnp.int32, (CHUNK, CHUNK), 0)
           ).astype(jnp.float32)
    carry = jnp.zeros((1, E), dtype=jnp.float32)
    for c in range(NCH):
        oh_c = onehot[c * CHUNK:(c + 1) * CHUNK]
        csum = jnp.dot(tri, oh_c, preferred_element_type=jnp.float32) + carry
        rank = jnp.sum(csum * oh_c, axis=1, keepdims=True) - 1.0
        pos_c = rank + rowoff[c * CHUNK:(c + 1) * CHUNK]
        pos_ref[c * CHUNK:(c + 1) * CHUNK, :] = pos_c.astype(jnp.int32)
        carry = csum[CHUNK - 1:CHUNK, :]
    i8 = (lax.broadcasted_iota(jnp.int32, (E, E), 0)
          == lax.broadcasted_iota(jnp.int32, (E, E), 1)).astype(jnp.float32)
    l8 = (lax.broadcasted_iota(jnp.int32, (E, E), 1)
          <= lax.broadcasted_iota(jnp.int32, (E, E), 0)).astype(jnp.float32)
    ones81 = jnp.ones((E, 1), dtype=jnp.float32)
    nt_sub = jnp.dot(i8 * ntiles, ones81,
                     preferred_element_type=jnp.float32)
    cnt_sub = jnp.dot(i8 * counts, ones81,
                      preferred_element_type=jnp.float32)
    cum_sub = jnp.dot(l8, nt_sub,
                      preferred_element_type=jnp.float32)
    eidx_sub = lax.broadcasted_iota(jnp.int32, (E, 1), 0).astype(jnp.float32)
    lue = jnp.max(jnp.where(cnt_sub > 0, eidx_sub, 0.0),
                  axis=0, keepdims=True)
    nu = jnp.max(cum_sub, axis=0, keepdims=True)
    ti = lax.broadcasted_iota(jnp.int32, (E, MAX_TILES), 1).astype(jnp.float32)
    te = jnp.sum((cum_sub <= ti).astype(jnp.float32),
                 axis=0, keepdims=True)
    te = jnp.minimum(te, lue)
    meta = jnp.concatenate(
        [te, jnp.broadcast_to(nu, (1, E))], axis=1)
    meta_ref[...] = meta.astype(jnp.int32)


def _mlp_body(m_ref, x_ref, w1_ref, b1_ref, w2_ref, b2_ref, o_ref):
    i = pl.program_id(0)
    hc = pl.program_id(1)

    @pl.when(i < m_ref[0, MAX_TILES])
    def _():
        xt = x_ref[...]
        h = jnp.dot(xt, w1_ref[0],
                    preferred_element_type=jnp.float32) + b1_ref[0]
        h = 0.5 * h * (1.0 + lax.erf(h * _INV_SQRT2))
        part = jnp.dot(h, w2_ref[0], preferred_element_type=jnp.float32)

        @pl.when(hc == 0)
        def _():
            o_ref[...] = part + b2_ref[0]

        @pl.when(hc != 0)
        def _():
            o_ref[...] += part


_NC, _NS = 2, 16
_NW = _NC * _NS
_CPW = T // _NW


@functools.cache
def _sc_kernels():
    mesh = plsc.VectorSubcoreMesh(core_axis_name="c", subcore_axis_name="s")

    @functools.partial(
        pl.kernel, mesh=mesh,
        out_type=jax.ShapeDtypeStruct((P, D), jnp.float32),
        scratch_types=[
            pltpu.VMEM((_CPW,), jnp.int32),
            pltpu.VMEM((_CPW, D), jnp.float32),
            pltpu.SemaphoreType.DMA,
        ],
    )
    def _sc_scatter(x_hbm, pos_hbm, xp_hbm, idx_v, rows_v, sem):
        wid = lax.axis_index("s") * _NC + lax.axis_index("c")
        base = wid * _CPW
        pltpu.sync_copy(pos_hbm.at[pl.ds(base, _CPW)], idx_v)
        pltpu.sync_copy(x_hbm.at[0, pl.ds(base, _CPW)], rows_v)
        pltpu.async_copy(rows_v, xp_hbm.at[idx_v], sem).wait()

    @functools.partial(
        pl.kernel, mesh=mesh,
        out_type=jax.ShapeDtypeStruct((B, S, D), jnp.float32),
        scratch_types=[
            pltpu.VMEM((_CPW,), jnp.int32),
            pltpu.VMEM((_CPW, D), jnp.float32),
            pltpu.SemaphoreType.DMA,
        ],
    )
    def _sc_gather(op_hbm, pos_hbm, out_hbm, idx_v, rows_v, sem):
        wid = lax.axis_index("s") * _NC + lax.axis_index("c")
        base = wid * _CPW
        pltpu.sync_copy(pos_hbm.at[pl.ds(base, _CPW)], idx_v)
        pltpu.async_copy(op_hbm.at[idx_v], rows_v, sem).wait()
        pltpu.sync_copy(rows_v, out_hbm.at[0, pl.ds(base, _CPW)])

    return _sc_scatter, _sc_gather


def kernel(x, Wg, bg, W1, b1, W2, b2):
    pos2d, meta = pl.pallas_call(
        _gate_body,
        out_shape=(
            jax.ShapeDtypeStruct((T, 1), jnp.int32),
            jax.ShapeDtypeStruct((1, MAX_TILES + E), jnp.int32),
        ),
    )(x, Wg, bg.reshape(1, E))
    pos = pos2d.reshape(T)

    sc_scatter, sc_gather = _sc_kernels()
    xp = sc_scatter(x, pos)

    grid_spec = pltpu.PrefetchScalarGridSpec(
        num_scalar_prefetch=1,
        grid=(MAX_TILES, KH),
        in_specs=[
            pl.BlockSpec(
                (TILE, D),
                lambda i, hc, m: (jnp.minimum(i, m[0, MAX_TILES] - 1), 0)),
            pl.BlockSpec((1, D, HC), lambda i, hc, m: (m[0, i], 0, hc)),
            pl.BlockSpec((1, 1, HC), lambda i, hc, m: (m[0, i], 0, hc)),
            pl.BlockSpec((1, HC, D), lambda i, hc, m: (m[0, i], hc, 0)),
            pl.BlockSpec((1, 1, D), lambda i, hc, m: (m[0, i], 0, 0)),
        ],
        out_specs=pl.BlockSpec(
            (TILE, D),
            lambda i, hc, m: (jnp.minimum(i, m[0, MAX_TILES] - 1), 0)),
    )
    op = pl.pallas_call(
        _mlp_body,
        grid_spec=grid_spec,
        out_shape=jax.ShapeDtypeStruct((P, D), jnp.float32),
    )(meta, xp, W1, b1.reshape(E, 1, H), W2, b2.reshape(E, 1, D))

    out = sc_gather(op, pos)
    return out, jnp.zeros((), dtype=jnp.float32)


# speedup vs baseline: 1.1697x; 1.1697x over previous
import functools

import jax
import jax.numpy as jnp
from jax import lax
from jax.experimental import pallas as pl
from jax.experimental.pallas import tpu as pltpu
from jax.experimental.pallas import tpu_sc as plsc

B, S, D, E, H = 1, 2048, 768, 8, 1024
T = B * S
TILE = 256
MAX_TILES = 16
P = MAX_TILES * TILE
KH = 1
HC = H // KH
CHUNK = 128
NCH = T // CHUNK

_INV_SQRT2 = 0.7071067811865476


def _gate_body(x_ref, wg_ref, bg_ref, pos_ref, meta_ref):
    xf = x_ref[0]
    scores = jnp.dot(xf, wg_ref[...],
                     preferred_element_type=jnp.float32) + bg_ref[...]
    m = jnp.max(scores, axis=1, keepdims=True)
    eidx = lax.broadcasted_iota(jnp.int32, (T, E), 1)
    sel = jnp.min(jnp.where(scores >= m, eidx, E), axis=1,
                  keepdims=True)
    onehot = (eidx == sel).astype(jnp.float32)
    counts = jnp.sum(onehot, axis=0, keepdims=True)
    ntiles = jnp.floor((counts + (TILE - 1)) * (1.0 / TILE))
    lt = (eidx < sel).astype(jnp.float32)
    rowoff = TILE * jnp.sum(ntiles * lt, axis=1, keepdims=True)
    tri = (lax.broadcasted_iota(jnp.int32, (CHUNK, CHUNK), 1)
           <= lax.broadcasted_iota(jnp.int32, (CHUNK, CHUNK), 0)
           ).astype(jnp.float32)
    carry = jnp.zeros((1, E), dtype=jnp.float32)
    for c in range(NCH):
        oh_c = onehot[c * CHUNK:(c + 1) * CHUNK]
        csum = jnp.dot(tri, oh_c, preferred_element_type=jnp.float32) + carry
        rank = jnp.sum(csum * oh_c, axis=1, keepdims=True) - 1.0
        pos_c = rank + rowoff[c * CHUNK:(c + 1) * CHUNK]
        pos_ref[c * CHUNK:(c + 1) * CHUNK, :] = pos_c.astype(jnp.int32)
        carry = csum[CHUNK - 1:CHUNK, :]
    i8 = (lax.broadcasted_iota(jnp.int32, (E, E), 0)
          == lax.broadcasted_iota(jnp.int32, (E, E), 1)).astype(jnp.float32)
    l8 = (lax.broadcasted_iota(jnp.int32, (E, E), 1)
          <= lax.broadcasted_iota(jnp.int32, (E, E), 0)).astype(jnp.float32)
    ones81 = jnp.ones((E, 1), dtype=jnp.float32)
    nt_sub = jnp.dot(i8 * ntiles, ones81,
                     preferred_element_type=jnp.float32)
    cnt_sub = jnp.dot(i8 * counts, ones81,
                      preferred_element_type=jnp.float32)
    cum_sub = jnp.dot(l8, nt_sub,
                      preferred_element_type=jnp.float32)
    eidx_sub = lax.broadcasted_iota(jnp.int32, (E, 1), 0).astype(jnp.float32)
    lue = jnp.max(jnp.where(cnt_sub > 0, eidx_sub, 0.0),
                  axis=0, keepdims=True)
    nu = jnp.max(cum_sub, axis=0, keepdims=True)
    ti = lax.broadcasted_iota(jnp.int32, (E, MAX_TILES), 1).astype(jnp.float32)
    te = jnp.sum((cum_sub <= ti).astype(jnp.float32),
                 axis=0, keepdims=True)
    te = jnp.minimum(te, lue)
    prev = jnp.concatenate([jnp.full((1, 1), -1.0, jnp.float32),
                            te[:, :MAX_TILES - 1]], axis=1)
    ch = (te != prev).astype(jnp.float32)
    lt16 = (lax.broadcasted_iota(jnp.int32, (MAX_TILES, MAX_TILES), 0)
            <= lax.broadcasted_iota(jnp.int32, (MAX_TILES, MAX_TILES), 1)
            ).astype(jnp.float32)
    dense = jnp.dot(ch, lt16, preferred_element_type=jnp.float32)
    slot = dense - 3.0 * jnp.floor(dense * (1.0 / 3.0))
    meta = jnp.concatenate(
        [te, slot, jnp.broadcast_to(nu, (1, E))], axis=1)
    meta_ref[...] = meta.astype(jnp.int32)


def _mlp_body(m_ref, x_ref, w1_ref, b1_ref, w2_ref, b2_ref, o_ref,
              w1_buf, w2_buf, sem1, sem2):
    i = pl.program_id(0)
    nu = m_ref[0, 2 * MAX_TILES]

    def start_w(e, s):
        pltpu.make_async_copy(w1_ref.at[e], w1_buf.at[s], sem1.at[s]).start()
        pltpu.make_async_copy(w2_ref.at[e], w2_buf.at[s], sem2.at[s]).start()

    def wait_w(s):
        pltpu.make_async_copy(w1_ref.at[0], w1_buf.at[s], sem1.at[s]).wait()
        pltpu.make_async_copy(w2_ref.at[0], w2_buf.at[s], sem2.at[s]).wait()

    @pl.when(i < nu)
    def _():
        te_i = m_ref[0, i]
        sl_i = m_ref[0, MAX_TILES + i]
        i1 = jnp.minimum(i + 1, MAX_TILES - 1)
        i2 = jnp.minimum(i + 2, MAX_TILES - 1)
        te_1, sl_1 = m_ref[0, i1], m_ref[0, MAX_TILES + i1]
        te_2, sl_2 = m_ref[0, i2], m_ref[0, MAX_TILES + i2]
        te_p = m_ref[0, jnp.maximum(i - 1, 0)]

        @pl.when(i == 0)
        def _():
            start_w(te_i, sl_i)

            @pl.when(te_1 != te_i)
            def _():
                start_w(te_1, sl_1)

        @pl.when((i == 0) | (te_i != te_p))
        def _():
            wait_w(sl_i)

        @pl.when(te_2 != te_1)
        def _():
            start_w(te_2, sl_2)

        xt = x_ref[...]
        h = jnp.dot(xt, w1_buf[sl_i],
                    preferred_element_type=jnp.float32) + b1_ref[0]
        h = 0.5 * h * (1.0 + lax.erf(h * _INV_SQRT2))
        o_ref[...] = jnp.dot(h, w2_buf[sl_i],
                             preferred_element_type=jnp.float32) + b2_ref[0]


_NC, _NS = 2, 16
_NW = _NC * _NS
_CPW = T // _NW


@functools.cache
def _sc_kernels():
    mesh = plsc.VectorSubcoreMesh(core_axis_name="c", subcore_axis_name="s")

    @functools.partial(
        pl.kernel, mesh=mesh,
        out_type=jax.ShapeDtypeStruct((P, D), jnp.float32),
        scratch_types=[
            pltpu.VMEM((_CPW,), jnp.int32),
            pltpu.VMEM((_CPW, D), jnp.float32),
            pltpu.SemaphoreType.DMA,
        ],
    )
    def _sc_scatter(x_hbm, pos_hbm, xp_hbm, idx_v, rows_v, sem):
        wid = lax.axis_index("s") * _NC + lax.axis_index("c")
        base = wid * _CPW
        pltpu.sync_copy(pos_hbm.at[pl.ds(base, _CPW)], idx_v)
        pltpu.sync_copy(x_hbm.at[0, pl.ds(base, _CPW)], rows_v)
        pltpu.async_copy(rows_v, xp_hbm.at[idx_v], sem).wait()

    @functools.partial(
        pl.kernel, mesh=mesh,
        out_type=jax.ShapeDtypeStruct((B, S, D), jnp.float32),
        scratch_types=[
            pltpu.VMEM((_CPW,), jnp.int32),
            pltpu.VMEM((_CPW, D), jnp.float32),
            pltpu.SemaphoreType.DMA,
        ],
    )
    def _sc_gather(op_hbm, pos_hbm, out_hbm, idx_v, rows_v, sem):
        wid = lax.axis_index("s") * _NC + lax.axis_index("c")
        base = wid * _CPW
        pltpu.sync_copy(pos_hbm.at[pl.ds(base, _CPW)], idx_v)
        pltpu.async_copy(op_hbm.at[idx_v], rows_v, sem).wait()
        pltpu.sync_copy(rows_v, out_hbm.at[0, pl.ds(base, _CPW)])

    return _sc_scatter, _sc_gather


def kernel(x, Wg, bg, W1, b1, W2, b2):
    pos2d, meta = pl.pallas_call(
        _gate_body,
        out_shape=(
            jax.ShapeDtypeStruct((T, 1), jnp.int32),
            jax.ShapeDtypeStruct((1, 2 * MAX_TILES + E), jnp.int32),
        ),
    )(x, Wg, bg.reshape(1, E))
    pos = pos2d.reshape(T)

    sc_scatter, sc_gather = _sc_kernels()
    xp = sc_scatter(x, pos)

    grid_spec = pltpu.PrefetchScalarGridSpec(
        num_scalar_prefetch=1,
        grid=(MAX_TILES,),
        in_specs=[
            pl.BlockSpec(
                (TILE, D),
                lambda i, m: (jnp.minimum(i, m[0, 2 * MAX_TILES] - 1), 0)),
            pl.BlockSpec(memory_space=pl.ANY),
            pl.BlockSpec((1, 1, H), lambda i, m: (m[0, i], 0, 0)),
            pl.BlockSpec(memory_space=pl.ANY),
            pl.BlockSpec((1, 1, D), lambda i, m: (m[0, i], 0, 0)),
        ],
        out_specs=pl.BlockSpec(
            (TILE, D),
            lambda i, m: (jnp.minimum(i, m[0, 2 * MAX_TILES] - 1), 0)),
        scratch_shapes=[
            pltpu.VMEM((3, D, H), jnp.float32),
            pltpu.VMEM((3, H, D), jnp.float32),
            pltpu.SemaphoreType.DMA((3,)),
            pltpu.SemaphoreType.DMA((3,)),
        ],
    )
    op = pl.pallas_call(
        _mlp_body,
        grid_spec=grid_spec,
        out_shape=jax.ShapeDtypeStruct((P, D), jnp.float32),
    )(meta, xp, W1, b1.reshape(E, 1, H), W2, b2.reshape(E, 1, D))

    out = sc_gather(op, pos)
    return out, jnp.zeros((), dtype=jnp.float32)
